# baseline (speedup 1.0000x reference)
import functools

import jax
import jax.numpy as jnp
from jax import lax
from jax.experimental import pallas as pl
from jax.experimental.pallas import tpu as pltpu
from jax.experimental.pallas import tpu_sc as plsc

_B, _N, _NT, _M, _K, _H = 1, 10000, 2, 32, 32, 50
_RMIN, _RMAX = 0.5, 6.0
_INV_DR = 1.0 / (_RMAX - _RMIN)
_LANES = 128
_NP = 10240
_GRID = _NP // _LANES
_NSLOT = _NT * _M
_E = _NSLOT * _NP
_NW = 32
_EW = _E // _NW
_CR = _NP // _LANES
_N1P = 10240
_SLICE = _N1P // 16


def _dense_body(d_ref, aux_ref, j_ref, w1s_ref, w2bd_ref, w2bdb_ref,
                w1bs_ref, par2_ref, par_ref, ef_ref, g_ref, idx_ref, et_ref,
                feat_ref):
    f32 = jnp.float32
    i = pl.program_id(0)
    dx = d_ref[0]
    dy = d_ref[1]
    dz = d_ref[2]
    r2 = dx * dx + dy * dy + dz * dz + 1e-12
    r = jnp.sqrt(r2)
    u = (r - _RMIN) * _INV_DR
    x = jnp.clip(2.0 * u - 1.0, -1.0, 1.0)
    inside = jnp.logical_and(r > _RMIN, r < _RMAX)
    insf = inside.astype(f32)
    uc = jnp.clip(u, 0.0, 1.0)
    fc = jnp.where(r < _RMAX, 0.5 * (jnp.cos(jnp.pi * uc) + 1.0), 0.0)

    rows = lax.broadcasted_iota(jnp.int32, (_NSLOT, _LANES), 0)
    mloc = jnp.bitwise_and(rows, _M - 1)
    nn0 = jnp.broadcast_to(aux_ref[1:2, :], (_NSLOT, _LANES))
    nn1 = jnp.broadcast_to(aux_ref[2:3, :], (_NSLOT, _LANES))
    nnb = jnp.where(rows < _M, nn0, nn1)
    vm = (mloc < nnb).astype(f32)
    w = fc * vm

    tkm1 = None
    tk = jnp.ones_like(x)
    for k in range(_K):
        wt = w * tk
        feat_ref[k:k + 1, :] = jnp.sum(wt[0:_M, :], axis=0, keepdims=True)
        feat_ref[_K + k:_K + k + 1, :] = jnp.sum(wt[_M:, :], axis=0,
                                                 keepdims=True)
        tn = x if k == 0 else 2.0 * x * tk - tkm1
        tkm1, tk = tk, tn

    mean_c = jnp.broadcast_to(par_ref[:, 8:9], (_NSLOT, _LANES))
    scale_c = jnp.broadcast_to(par_ref[:, 9:10], (_NSLOT, _LANES))
    fs = (feat_ref[...] - mean_c) * scale_c
    it = aux_ref[0:1, :]

    rows2 = lax.broadcasted_iota(jnp.int32, (2 * _NSLOT, _LANES), 0)
    trow = (rows2 >= _NSLOT).astype(jnp.int32)
    itb = jnp.broadcast_to(it, (2 * _NSLOT, _LANES))
    msk = (itb == trow).astype(f32)
    b1c = jnp.broadcast_to(par2_ref[:, 0:1], (2 * _NSLOT, _LANES))
    b2c = jnp.broadcast_to(par2_ref[:, 1:2], (2 * _NSLOT, _LANES))
    w3c = jnp.broadcast_to(par2_ref[:, 2:3], (2 * _NSLOT, _LANES))
    z1 = jnp.dot(w1s_ref[...], fs, preferred_element_type=f32) + b1c
    h1 = jnp.tanh(z1)
    z2 = jnp.dot(w2bd_ref[...], h1, preferred_element_type=f32) + b2c
    th2 = jnp.tanh(z2)
    h2 = th2 + h1
    dh2 = w3c * msk
    b3a = jnp.broadcast_to(par2_ref[0:1, 3:4], (1, _LANES))
    b3b = jnp.broadcast_to(par2_ref[1:2, 3:4], (1, _LANES))
    ei = (jnp.sum(dh2 * h2, axis=0, keepdims=True) +
          jnp.where(it == 0, b3a, b3b))
    dz2 = dh2 * (1.0 - th2 * th2)
    dh1 = dh2 + jnp.dot(w2bdb_ref[...], dz2, preferred_element_type=f32)
    dz1 = dh1 * (1.0 - h1 * h1)
    dfs = jnp.dot(w1bs_ref[...], dz1, preferred_element_type=f32)
    dft = dfs * scale_c

    col = lax.broadcasted_iota(jnp.int32, (1, _LANES), 1)
    valid = ((i * _LANES + col) < _N).astype(f32)
    ei = ei * valid

    kv = jnp.bitwise_and(rows, _K - 1).astype(f32)
    dftk = dft * kv
    fcp = jnp.where(inside, -0.5 * jnp.pi * jnp.sin(jnp.pi * uc) * _INV_DR,
                    0.0)
    x2 = 2.0 * x
    sgx = jnp.zeros((1, _LANES), f32)
    sgy = jnp.zeros((1, _LANES), f32)
    sgz = jnp.zeros((1, _LANES), f32)
    for h in range(_NT):
        lo = h * _M
        xh = x[lo:lo + _M, :]
        x2h = x2[lo:lo + _M, :]
        z = jnp.zeros((_M, _LANES), f32)
        bT1, bT2 = z, z
        for k in range(_K - 1, -1, -1):
            c = jnp.broadcast_to(dft[h * _K + k:h * _K + k + 1, :],
                                 (_M, _LANES))
            bT1, bT2 = c + x2h * bT1 - bT2, bT1
        pacc = bT1 - xh * bT2
        bU1, bU2 = z, z
        for k in range(_K - 1, 0, -1):
            d = jnp.broadcast_to(dftk[h * _K + k:h * _K + k + 1, :],
                                 (_M, _LANES))
            bU1, bU2 = d + x2h * bU1 - bU2, bU1
        dacc = bU1
        dedr = (dacc * (2.0 * _INV_DR) * insf[lo:lo + _M, :] *
                fc[lo:lo + _M, :] + pacc * fcp[lo:lo + _M, :])
        gs = dedr * vm[lo:lo + _M, :] / r[lo:lo + _M, :]
        gx = gs * dx[lo:lo + _M, :]
        gy = gs * dy[lo:lo + _M, :]
        gz = gs * dz[lo:lo + _M, :]
        g_ref[0, lo:lo + _M, :] = gx
        g_ref[1, lo:lo + _M, :] = gy
        g_ref[2, lo:lo + _M, :] = gz
        sgx = sgx + jnp.sum(gx, axis=0, keepdims=True)
        sgy = sgy + jnp.sum(gy, axis=0, keepdims=True)
        sgz = sgz + jnp.sum(gz, axis=0, keepdims=True)
    dump = _N + jnp.broadcast_to(col, (_NSLOT, _LANES))
    idx_ref[...] = jnp.where(vm > 0.0, j_ref[...], dump)
    ef_ref[0:1, :] = ei
    ef_ref[1:2, :] = sgx
    ef_ref[2:3, :] = sgy
    ef_ref[3:4, :] = sgz
    ef_ref[4:8, :] = jnp.zeros((4, _LANES), f32)

    @pl.when(i == 0)
    def _():
        et_ref[...] = jnp.zeros((8, _LANES), f32)

    et_ref[0:1, :] = et_ref[0:1, :] + ei


def _dense_call(dxt, aux, jt, w1s, w2bd, w2bdb, w1bs, par2, par):
    return pl.pallas_call(
        _dense_body,
        grid=(_GRID,),
        in_specs=[
            pl.BlockSpec((3, _NSLOT, _LANES), lambda i: (0, 0, i)),
            pl.BlockSpec((8, _LANES), lambda i: (0, i)),
            pl.BlockSpec((_NSLOT, _LANES), lambda i: (0, i)),
            pl.BlockSpec((128, 64), lambda i: (0, 0)),
            pl.BlockSpec((128, 128), lambda i: (0, 0)),
            pl.BlockSpec((128, 128), lambda i: (0, 0)),
            pl.BlockSpec((64, 128), lambda i: (0, 0)),
            pl.BlockSpec((128, 8), lambda i: (0, 0)),
            pl.BlockSpec((64, _LANES), lambda i: (0, 0)),
        ],
        out_specs=[
            pl.BlockSpec((8, _LANES), lambda i: (0, i)),
            pl.BlockSpec((3, _NSLOT, _LANES), lambda i: (0, 0, i)),
            pl.BlockSpec((_NSLOT, _LANES), lambda i: (0, i)),
            pl.BlockSpec((8, _LANES), lambda i: (0, 0)),
        ],
        out_shape=[
            jax.ShapeDtypeStruct((8, _NP), jnp.float32),
            jax.ShapeDtypeStruct((3, _NSLOT, _NP), jnp.float32),
            jax.ShapeDtypeStruct((_NSLOT, _NP), jnp.int32),
            jax.ShapeDtypeStruct((8, _LANES), jnp.float32),
        ],
        scratch_shapes=[pltpu.VMEM((_NSLOT, _LANES), jnp.float32)],
    )(dxt, aux, jt, w1s, w2bd, w2bdb, w1bs, par2, par)


_PD = 8


_RW = _NSLOT // _NW


def _scatter_body(g_hbm, idx_hbm, out_hbm, idxv, gv0, gv1, gv2, zb,
                  acc0, acc1, acc2, sem):
    c = lax.axis_index("c")
    s = lax.axis_index("s")
    wid = c * 16 + s
    row = wid * _RW
    z16 = jnp.zeros((16,), jnp.float32)

    pltpu.sync_copy(idx_hbm.at[pl.ds(row, _RW)], idxv)
    pltpu.sync_copy(g_hbm.at[pl.ds(0 * _NSLOT + row, _RW)], gv0)
    pltpu.sync_copy(g_hbm.at[pl.ds(1 * _NSLOT + row, _RW)], gv1)
    pltpu.sync_copy(g_hbm.at[pl.ds(2 * _NSLOT + row, _RW)], gv2)

    def zloop(i, carry):
        zb[pl.ds(16 * i, 16)] = z16
        return carry

    lax.fori_loop(0, _SLICE // 16, zloop, 0)
    zoff = s * _SLICE
    pltpu.sync_copy(zb, acc0.at[pl.ds(zoff, _SLICE)])
    pltpu.sync_copy(zb, acc1.at[pl.ds(zoff, _SLICE)])
    pltpu.sync_copy(zb, acc2.at[pl.ds(zoff, _SLICE)])
    plsc.subcore_barrier()

    for r in range(_RW):
        def fire(o, r=r):
            off = o * _LANES
            iv = idxv.at[r, pl.ds(off, _LANES)]
            pltpu.async_copy(gv0.at[r, pl.ds(off, _LANES)], acc0.at[iv],
                             sem, add=True)
            pltpu.async_copy(gv1.at[r, pl.ds(off, _LANES)], acc1.at[iv],
                             sem, add=True)
            pltpu.async_copy(gv2.at[r, pl.ds(off, _LANES)], acc2.at[iv],
                             sem, add=True)

        def drain(o, r=r):
            off = o * _LANES
            iv = idxv.at[r, pl.ds(off, _LANES)]
            pltpu.make_async_copy(gv0.at[r, pl.ds(off, _LANES)],
                                  acc0.at[iv], sem).wait()
            pltpu.make_async_copy(gv1.at[r, pl.ds(off, _LANES)],
                                  acc1.at[iv], sem).wait()
            pltpu.make_async_copy(gv2.at[r, pl.ds(off, _LANES)],
                                  acc2.at[iv], sem).wait()

        def inner(o, carry):
            fire(o)

            @pl.when(o >= _PD)
            def _():
                drain(o - _PD)

            return carry

        lax.fori_loop(0, _CR, inner, 0)

        def dloop(o, carry):
            drain(o)
            return carry

        lax.fori_loop(_CR - _PD, _CR, dloop, 0)
    plsc.subcore_barrier()

    pltpu.sync_copy(acc0.at[pl.ds(zoff, _SLICE)],
                    out_hbm.at[pl.ds(c * 3 * _N1P + 0 * _N1P + zoff, _SLICE)])
    pltpu.sync_copy(acc1.at[pl.ds(zoff, _SLICE)],
                    out_hbm.at[pl.ds(c * 3 * _N1P + 1 * _N1P + zoff, _SLICE)])
    pltpu.sync_copy(acc2.at[pl.ds(zoff, _SLICE)],
                    out_hbm.at[pl.ds(c * 3 * _N1P + 2 * _N1P + zoff, _SLICE)])


def _scatter_call(g2d, idx2d):
    mesh = plsc.VectorSubcoreMesh(core_axis_name="c", subcore_axis_name="s")
    f = functools.partial(
        pl.kernel,
        mesh=mesh,
        out_type=jax.ShapeDtypeStruct((2 * 3 * _N1P,), jnp.float32),
        scratch_types=[
            pltpu.VMEM((_RW, _NP), jnp.int32),
            pltpu.VMEM((_RW, _NP), jnp.float32),
            pltpu.VMEM((_RW, _NP), jnp.float32),
            pltpu.VMEM((_RW, _NP), jnp.float32),
            pltpu.VMEM((_SLICE,), jnp.float32),
            pltpu.VMEM_SHARED((_N1P,), jnp.float32),
            pltpu.VMEM_SHARED((_N1P,), jnp.float32),
            pltpu.VMEM_SHARED((_N1P,), jnp.float32),
            pltpu.SemaphoreType.DMA,
        ],
    )(_scatter_body)
    return f(g2d, idx2d)


def kernel(list_neigh, Imagetype_map, atom_type, ImageDR, num_neigh, nghost,
           scaler_scale, scaler_mean, W1, b1, W2, b2, W3, b3, ener_shift):
    f32 = jnp.float32
    pad = _NP - _N

    dxyz = ImageDR[0, :, :, 1:4].astype(f32)
    dxt = jnp.transpose(dxyz, (2, 1, 0))
    dxt = jnp.pad(dxt, ((0, 0), (0, 0), (0, pad)))

    aux = jnp.zeros((8, _NP), jnp.int32)
    aux = aux.at[0, :_N].set(Imagetype_map)
    aux = aux.at[1, :_N].set(num_neigh[0, :, 0])
    aux = aux.at[2, :_N].set(num_neigh[0, :, 1])

    jt = jnp.pad(jnp.transpose(list_neigh[0].reshape(_N, _NSLOT), (1, 0)),
                 ((0, 0), (0, pad)))

    hp = 64 - _H
    w1p = jnp.pad(W1, ((0, 0), (0, 0), (0, hp)))
    b1p = jnp.pad(b1, ((0, 0), (0, hp)))
    w2p = jnp.pad(W2, ((0, 0), (0, hp), (0, hp)))
    b2p = jnp.pad(b2, ((0, 0), (0, hp)))
    w3p = jnp.pad(W3[..., 0], ((0, 0), (0, hp)))
    b3s = b3[:, 0] + ener_shift[:, 0]

    par = jnp.zeros((64, _LANES), f32)
    par = par.at[:, 8].set(scaler_mean).at[:, 9].set(scaler_scale)

    w1f = jnp.transpose(w1p, (0, 2, 1))
    w2f = jnp.transpose(w2p, (0, 2, 1))
    w1s = jnp.concatenate([w1f[0], w1f[1]], axis=0)
    w2bd = (jnp.zeros((128, 128), f32)
            .at[:64, :64].set(w2f[0]).at[64:, 64:].set(w2f[1]))
    w2bdb = (jnp.zeros((128, 128), f32)
             .at[:64, :64].set(w2p[0]).at[64:, 64:].set(w2p[1]))
    w1bs = jnp.concatenate([w1p[0], w1p[1]], axis=1)
    par2 = jnp.zeros((128, 8), f32)
    par2 = par2.at[:64, 0].set(b1p[0]).at[64:, 0].set(b1p[1])
    par2 = par2.at[:64, 1].set(b2p[0]).at[64:, 1].set(b2p[1])
    par2 = par2.at[:64, 2].set(w3p[0]).at[64:, 2].set(w3p[1])
    par2 = par2.at[0, 3].set(b3s[0]).at[1, 3].set(b3s[1])

    ef, g, idx, et = _dense_call(dxt, aux, jt, w1s, w2bd, w2bdb, w1bs, par2,
                                 par)

    facc = _scatter_call(g.reshape(3 * _NSLOT, _NP), idx).reshape(2, 3, _N1P)

    fsc = (facc[0] + facc[1])[:, :_N]
    force = jnp.transpose(ef[1:4, :_N] - fsc, (1, 0))[None]
    etot = jnp.sum(et[0, :]).reshape(1, 1)
    ei_out = ef[0:1, :_N]
    return etot, ei_out, force


# speedup vs baseline: 1.0103x; 1.0103x over previous
import functools

import jax
import jax.numpy as jnp
from jax import lax
from jax.experimental import pallas as pl
from jax.experimental.pallas import tpu as pltpu
from jax.experimental.pallas import tpu_sc as plsc

_B, _N, _NT, _M, _K, _H = 1, 10000, 2, 32, 32, 50
_RMIN, _RMAX = 0.5, 6.0
_INV_DR = 1.0 / (_RMAX - _RMIN)
_LANES = 128
_NP = 10240
_GRID = _NP // _LANES
_NSLOT = _NT * _M
_E = _NSLOT * _NP
_NW = 32
_EW = _E // _NW
_CR = _NP // _LANES
_N1P = 10240
_SLICE = _N1P // 16


def _dense_body(d_ref, aux_ref, j_ref, w1s_ref, w2bd_ref, w2bdb_ref,
                w1bs_ref, par2_ref, par_ref, ef_ref, g_ref, idx_ref, et_ref,
                feat_ref):
    f32 = jnp.float32
    i = pl.program_id(0)
    dx = d_ref[0]
    dy = d_ref[1]
    dz = d_ref[2]
    r2 = dx * dx + dy * dy + dz * dz + 1e-12
    r = jnp.sqrt(r2)
    u = (r - _RMIN) * _INV_DR
    x = jnp.clip(2.0 * u - 1.0, -1.0, 1.0)
    inside = jnp.logical_and(r > _RMIN, r < _RMAX)
    insf = inside.astype(f32)
    uc = jnp.clip(u, 0.0, 1.0)
    fc = jnp.where(r < _RMAX, 0.5 * (jnp.cos(jnp.pi * uc) + 1.0), 0.0)

    rows = lax.broadcasted_iota(jnp.int32, (_NSLOT, _LANES), 0)
    mloc = jnp.bitwise_and(rows, _M - 1)
    nn0 = jnp.broadcast_to(aux_ref[1:2, :], (_NSLOT, _LANES))
    nn1 = jnp.broadcast_to(aux_ref[2:3, :], (_NSLOT, _LANES))
    nnb = jnp.where(rows < _M, nn0, nn1)
    vm = (mloc < nnb).astype(f32)
    w = fc * vm

    tkm1 = None
    tk = jnp.ones_like(x)
    for k in range(_K):
        wt = w * tk
        feat_ref[k:k + 1, :] = jnp.sum(wt[0:_M, :], axis=0, keepdims=True)
        feat_ref[_K + k:_K + k + 1, :] = jnp.sum(wt[_M:, :], axis=0,
                                                 keepdims=True)
        tn = x if k == 0 else 2.0 * x * tk - tkm1
        tkm1, tk = tk, tn

    mean_c = jnp.broadcast_to(par_ref[:, 8:9], (_NSLOT, _LANES))
    scale_c = jnp.broadcast_to(par_ref[:, 9:10], (_NSLOT, _LANES))
    fs = (feat_ref[...] - mean_c) * scale_c
    it = aux_ref[0:1, :]

    rows2 = lax.broadcasted_iota(jnp.int32, (2 * _NSLOT, _LANES), 0)
    trow = (rows2 >= _NSLOT).astype(jnp.int32)
    itb = jnp.broadcast_to(it, (2 * _NSLOT, _LANES))
    msk = (itb == trow).astype(f32)
    b1c = jnp.broadcast_to(par2_ref[:, 0:1], (2 * _NSLOT, _LANES))
    b2c = jnp.broadcast_to(par2_ref[:, 1:2], (2 * _NSLOT, _LANES))
    w3c = jnp.broadcast_to(par2_ref[:, 2:3], (2 * _NSLOT, _LANES))
    z1 = jnp.dot(w1s_ref[...], fs, preferred_element_type=f32) + b1c
    h1 = jnp.tanh(z1)
    z2 = jnp.dot(w2bd_ref[...], h1, preferred_element_type=f32) + b2c
    th2 = jnp.tanh(z2)
    h2 = th2 + h1
    dh2 = w3c * msk
    b3a = jnp.broadcast_to(par2_ref[0:1, 3:4], (1, _LANES))
    b3b = jnp.broadcast_to(par2_ref[1:2, 3:4], (1, _LANES))
    ei = (jnp.sum(dh2 * h2, axis=0, keepdims=True) +
          jnp.where(it == 0, b3a, b3b))
    dz2 = dh2 * (1.0 - th2 * th2)
    dh1 = dh2 + jnp.dot(w2bdb_ref[...], dz2, preferred_element_type=f32)
    dz1 = dh1 * (1.0 - h1 * h1)
    dfs = jnp.dot(w1bs_ref[...], dz1, preferred_element_type=f32)
    dft = dfs * scale_c

    col = lax.broadcasted_iota(jnp.int32, (1, _LANES), 1)
    valid = ((i * _LANES + col) < _N).astype(f32)
    ei = ei * valid

    pacc = jnp.zeros((_NSLOT, _LANES), f32)
    dacc = jnp.zeros((_NSLOT, _LANES), f32)
    tkm1 = None
    tk = jnp.ones_like(x)
    dtkm1 = None
    dtk = jnp.zeros_like(x)
    for k in range(_K):
        s0 = jnp.broadcast_to(dft[k:k + 1, :], (_NSLOT, _LANES))
        s1 = jnp.broadcast_to(dft[_K + k:_K + k + 1, :], (_NSLOT, _LANES))
        sb = jnp.where(rows < _M, s0, s1)
        pacc = pacc + sb * tk
        dacc = dacc + sb * dtk
        if k == 0:
            tn = x
            dtn = jnp.ones_like(x)
        else:
            tn = 2.0 * x * tk - tkm1
            dtn = 2.0 * tk + 2.0 * x * dtk - dtkm1
        tkm1, tk = tk, tn
        dtkm1, dtk = dtk, dtn

    fcp = jnp.where(inside, -0.5 * jnp.pi * jnp.sin(jnp.pi * uc) * _INV_DR,
                    0.0)
    dedr = dacc * (2.0 * _INV_DR) * insf * fc + pacc * fcp
    gs = dedr * vm / r
    gx = gs * dx
    gy = gs * dy
    gz = gs * dz
    g_ref[0, :, :] = gx
    g_ref[1, :, :] = gy
    g_ref[2, :, :] = gz
    sgx = jnp.sum(gx, axis=0, keepdims=True)
    sgy = jnp.sum(gy, axis=0, keepdims=True)
    sgz = jnp.sum(gz, axis=0, keepdims=True)
    dump = _N + jnp.broadcast_to(col, (_NSLOT, _LANES))
    idx_ref[...] = jnp.where(vm > 0.0, j_ref[...], dump)
    ef_ref[0:1, :] = ei
    ef_ref[1:2, :] = sgx
    ef_ref[2:3, :] = sgy
    ef_ref[3:4, :] = sgz
    ef_ref[4:8, :] = jnp.zeros((4, _LANES), f32)

    @pl.when(i == 0)
    def _():
        et_ref[...] = jnp.zeros((8, _LANES), f32)

    et_ref[0:1, :] = et_ref[0:1, :] + ei


def _dense_call(dxt, aux, jt, w1s, w2bd, w2bdb, w1bs, par2, par):
    return pl.pallas_call(
        _dense_body,
        grid=(_GRID,),
        in_specs=[
            pl.BlockSpec((3, _NSLOT, _LANES), lambda i: (0, 0, i)),
            pl.BlockSpec((8, _LANES), lambda i: (0, i)),
            pl.BlockSpec((_NSLOT, _LANES), lambda i: (0, i)),
            pl.BlockSpec((128, 64), lambda i: (0, 0)),
            pl.BlockSpec((128, 128), lambda i: (0, 0)),
            pl.BlockSpec((128, 128), lambda i: (0, 0)),
            pl.BlockSpec((64, 128), lambda i: (0, 0)),
            pl.BlockSpec((128, 8), lambda i: (0, 0)),
            pl.BlockSpec((64, _LANES), lambda i: (0, 0)),
        ],
        out_specs=[
            pl.BlockSpec((8, _LANES), lambda i: (0, i)),
            pl.BlockSpec((3, _NSLOT, _LANES), lambda i: (0, 0, i)),
            pl.BlockSpec((_NSLOT, _LANES), lambda i: (0, i)),
            pl.BlockSpec((8, _LANES), lambda i: (0, 0)),
        ],
        out_shape=[
            jax.ShapeDtypeStruct((8, _NP), jnp.float32),
            jax.ShapeDtypeStruct((3, _NSLOT, _NP), jnp.float32),
            jax.ShapeDtypeStruct((_NSLOT, _NP), jnp.int32),
            jax.ShapeDtypeStruct((8, _LANES), jnp.float32),
        ],
        scratch_shapes=[pltpu.VMEM((_NSLOT, _LANES), jnp.float32)],
    )(dxt, aux, jt, w1s, w2bd, w2bdb, w1bs, par2, par)


_PD = 8


_RW = _NSLOT // _NW


def _scatter_body(g_hbm, idx_hbm, out_hbm, idxv, gv0, gv1, gv2, zb,
                  acc0, acc1, acc2, sem):
    c = lax.axis_index("c")
    s = lax.axis_index("s")
    wid = c * 16 + s
    row = wid * _RW
    z16 = jnp.zeros((16,), jnp.float32)

    pltpu.sync_copy(idx_hbm.at[pl.ds(row, _RW)], idxv)
    pltpu.sync_copy(g_hbm.at[pl.ds(0 * _NSLOT + row, _RW)], gv0)
    pltpu.sync_copy(g_hbm.at[pl.ds(1 * _NSLOT + row, _RW)], gv1)
    pltpu.sync_copy(g_hbm.at[pl.ds(2 * _NSLOT + row, _RW)], gv2)

    def zloop(i, carry):
        zb[pl.ds(16 * i, 16)] = z16
        return carry

    lax.fori_loop(0, _SLICE // 16, zloop, 0)
    zoff = s * _SLICE
    pltpu.sync_copy(zb, acc0.at[pl.ds(zoff, _SLICE)])
    pltpu.sync_copy(zb, acc1.at[pl.ds(zoff, _SLICE)])
    pltpu.sync_copy(zb, acc2.at[pl.ds(zoff, _SLICE)])
    plsc.subcore_barrier()

    for r in range(_RW):
        def fire(o, r=r):
            off = o * _LANES
            iv = idxv.at[r, pl.ds(off, _LANES)]
            pltpu.async_copy(gv0.at[r, pl.ds(off, _LANES)], acc0.at[iv],
                             sem, add=True)
            pltpu.async_copy(gv1.at[r, pl.ds(off, _LANES)], acc1.at[iv],
                             sem, add=True)
            pltpu.async_copy(gv2.at[r, pl.ds(off, _LANES)], acc2.at[iv],
                             sem, add=True)

        def drain(o, r=r):
            off = o * _LANES
            iv = idxv.at[r, pl.ds(off, _LANES)]
            pltpu.make_async_copy(gv0.at[r, pl.ds(off, _LANES)],
                                  acc0.at[iv], sem).wait()
            pltpu.make_async_copy(gv1.at[r, pl.ds(off, _LANES)],
                                  acc1.at[iv], sem).wait()
            pltpu.make_async_copy(gv2.at[r, pl.ds(off, _LANES)],
                                  acc2.at[iv], sem).wait()

        def inner(o, carry):
            fire(o)

            @pl.when(o >= _PD)
            def _():
                drain(o - _PD)

            return carry

        lax.fori_loop(0, _CR, inner, 0)

        def dloop(o, carry):
            drain(o)
            return carry

        lax.fori_loop(_CR - _PD, _CR, dloop, 0)
    plsc.subcore_barrier()

    pltpu.sync_copy(acc0.at[pl.ds(zoff, _SLICE)],
                    out_hbm.at[pl.ds(c * 3 * _N1P + 0 * _N1P + zoff, _SLICE)])
    pltpu.sync_copy(acc1.at[pl.ds(zoff, _SLICE)],
                    out_hbm.at[pl.ds(c * 3 * _N1P + 1 * _N1P + zoff, _SLICE)])
    pltpu.sync_copy(acc2.at[pl.ds(zoff, _SLICE)],
                    out_hbm.at[pl.ds(c * 3 * _N1P + 2 * _N1P + zoff, _SLICE)])


def _scatter_call(g2d, idx2d):
    mesh = plsc.VectorSubcoreMesh(core_axis_name="c", subcore_axis_name="s")
    f = functools.partial(
        pl.kernel,
        mesh=mesh,
        out_type=jax.ShapeDtypeStruct((2 * 3 * _N1P,), jnp.float32),
        scratch_types=[
            pltpu.VMEM((_RW, _NP), jnp.int32),
            pltpu.VMEM((_RW, _NP), jnp.float32),
            pltpu.VMEM((_RW, _NP), jnp.float32),
            pltpu.VMEM((_RW, _NP), jnp.float32),
            pltpu.VMEM((_SLICE,), jnp.float32),
            pltpu.VMEM_SHARED((_N1P,), jnp.float32),
            pltpu.VMEM_SHARED((_N1P,), jnp.float32),
            pltpu.VMEM_SHARED((_N1P,), jnp.float32),
            pltpu.SemaphoreType.DMA,
        ],
    )(_scatter_body)
    return f(g2d, idx2d)


def kernel(list_neigh, Imagetype_map, atom_type, ImageDR, num_neigh, nghost,
           scaler_scale, scaler_mean, W1, b1, W2, b2, W3, b3, ener_shift):
    f32 = jnp.float32
    pad = _NP - _N

    dxyz = ImageDR[0, :, :, 1:4].astype(f32)
    dxt = jnp.transpose(dxyz, (2, 1, 0))
    dxt = jnp.pad(dxt, ((0, 0), (0, 0), (0, pad)))

    aux = jnp.zeros((8, _NP), jnp.int32)
    aux = aux.at[0, :_N].set(Imagetype_map)
    aux = aux.at[1, :_N].set(num_neigh[0, :, 0])
    aux = aux.at[2, :_N].set(num_neigh[0, :, 1])

    jt = jnp.pad(jnp.transpose(list_neigh[0].reshape(_N, _NSLOT), (1, 0)),
                 ((0, 0), (0, pad)))

    hp = 64 - _H
    w1p = jnp.pad(W1, ((0, 0), (0, 0), (0, hp)))
    b1p = jnp.pad(b1, ((0, 0), (0, hp)))
    w2p = jnp.pad(W2, ((0, 0), (0, hp), (0, hp)))
    b2p = jnp.pad(b2, ((0, 0), (0, hp)))
    w3p = jnp.pad(W3[..., 0], ((0, 0), (0, hp)))
    b3s = b3[:, 0] + ener_shift[:, 0]

    par = jnp.zeros((64, _LANES), f32)
    par = par.at[:, 8].set(scaler_mean).at[:, 9].set(scaler_scale)

    w1f = jnp.transpose(w1p, (0, 2, 1))
    w2f = jnp.transpose(w2p, (0, 2, 1))
    w1s = jnp.concatenate([w1f[0], w1f[1]], axis=0)
    w2bd = (jnp.zeros((128, 128), f32)
            .at[:64, :64].set(w2f[0]).at[64:, 64:].set(w2f[1]))
    w2bdb = (jnp.zeros((128, 128), f32)
             .at[:64, :64].set(w2p[0]).at[64:, 64:].set(w2p[1]))
    w1bs = jnp.concatenate([w1p[0], w1p[1]], axis=1)
    par2 = jnp.zeros((128, 8), f32)
    par2 = par2.at[:64, 0].set(b1p[0]).at[64:, 0].set(b1p[1])
    par2 = par2.at[:64, 1].set(b2p[0]).at[64:, 1].set(b2p[1])
    par2 = par2.at[:64, 2].set(w3p[0]).at[64:, 2].set(w3p[1])
    par2 = par2.at[0, 3].set(b3s[0]).at[1, 3].set(b3s[1])

    ef, g, idx, et = _dense_call(dxt, aux, jt, w1s, w2bd, w2bdb, w1bs, par2,
                                 par)

    facc = _scatter_call(g.reshape(3 * _NSLOT, _NP), idx).reshape(2, 3, _N1P)

    fsc = (facc[0] + facc[1])[:, :_N]
    force = jnp.transpose(ef[1:4, :_N] - fsc, (1, 0))[None]
    etot = jnp.sum(et[0, :]).reshape(1, 1)
    ei_out = ef[0:1, :_N]
    return etot, ei_out, force
